# baseline (speedup 1.0000x reference)
import functools

import jax
import jax.numpy as jnp
from jax import lax
from jax.experimental import pallas as pl
from jax.experimental.pallas import tpu as pltpu
from jax.experimental.pallas import tpu_sc as plsc

N = 10000
E = 160000
R = 8
D = 256
NR = N * R
NRP = 81920
NC, NS, L = 2, 16, 16
NW = NC * NS
EPW = E // NW
EPT = E // NS
KA = 80
NCH_A = EPT // KA
NGRP = (NCH_A + 7) // 8
KT = 64
NCH_T = (EPW + KT - 1) // KT
EPAD = NCH_T * KT
NB = 5
BN = N // NB

_MESH = dict(mesh=plsc.VectorSubcoreMesh(core_axis_name="c", subcore_axis_name="s",
                                         num_cores=NC, num_subcores=NS),
             compiler_params=pltpu.CompilerParams(needs_layout_passes=False))


def _wid():
    return lax.axis_index("s") * NC + lax.axis_index("c")


@functools.partial(
    pl.kernel,
    out_type=jax.ShapeDtypeStruct((NW, NRP), jnp.float32),
    scratch_types=[
        pltpu.VMEM((NRP,), jnp.float32),
        pltpu.VMEM((EPW + 16,), jnp.int32),
        pltpu.VMEM((EPW + 16,), jnp.int32),
    ],
    **_MESH,
)
def _hist_k(dst_hbm, et_hbm, out_hbm, hist_v, dst_v, et_v):
    wid = _wid()
    base = wid * EPW
    pltpu.sync_copy(dst_hbm.at[pl.ds(base, EPW)], dst_v.at[pl.ds(0, EPW)])
    pltpu.sync_copy(et_hbm.at[pl.ds(base, EPW)], et_v.at[pl.ds(0, EPW)])
    zero = jnp.zeros((L,), jnp.float32)

    def zbody(j, _):
        for u in range(16):
            hist_v[pl.ds((j * 16 + u) * L, L)] = zero
        return 0

    lax.fori_loop(0, NRP // (16 * L), zbody, 0)
    ones = jnp.ones((L,), jnp.float32)
    nfull = EPW // L

    def body(j, _):
        d16 = dst_v[pl.ds(j * L, L)]
        t16 = et_v[pl.ds(j * L, L)]
        seg = jnp.clip(d16 * R + t16, 0, NRP - 1)
        plsc.addupdate_scatter(hist_v, [seg], ones)
        return 0

    lax.fori_loop(0, nfull, body, 0)
    d16 = dst_v[pl.ds(nfull * L, L)]
    t16 = et_v[pl.ds(nfull * L, L)]
    seg = jnp.clip(d16 * R + t16, 0, NRP - 1)
    lane = lax.iota(jnp.int32, L)
    plsc.addupdate_scatter(hist_v, [seg], ones, mask=lane < (EPW - nfull * L))
    pltpu.sync_copy(hist_v, out_hbm.at[wid])


def _inv_body(h_ref, o_ref):
    cnt = jnp.sum(h_ref[...], axis=0)
    o_ref[...] = 1.0 / jnp.maximum(cnt, 1.0)


def _inv(hist3):
    return pl.pallas_call(
        _inv_body,
        grid=(10,),
        in_specs=[pl.BlockSpec((NW, 64, 128), lambda i: (0, i, 0))],
        out_specs=pl.BlockSpec((64, 128), lambda i: (i, 0)),
        out_shape=jax.ShapeDtypeStruct((640, 128), jnp.float32),
    )(hist3)


@functools.partial(
    pl.kernel,
    out_type=(
        jax.ShapeDtypeStruct((E,), jnp.float32),
        jax.ShapeDtypeStruct((E,), jnp.int32),
    ),
    scratch_types=[
        pltpu.VMEM((NRP,), jnp.float32),
        pltpu.VMEM((EPW + 16,), jnp.int32),
        pltpu.VMEM((EPW + 16,), jnp.int32),
        pltpu.VMEM((EPW + 16,), jnp.int32),
        pltpu.VMEM((EPW + 16,), jnp.float32),
        pltpu.VMEM((EPW + 16,), jnp.int32),
    ],
    **_MESH,
)
def _wg_k(src_hbm, dst_hbm, et_hbm, inv_hbm, w_hbm, gidx_hbm,
          inv_v, src_v, dst_v, et_v, w_v, gi_v):
    wid = _wid()
    base = wid * EPW
    pltpu.sync_copy(inv_hbm, inv_v)
    pltpu.sync_copy(src_hbm.at[pl.ds(base, EPW)], src_v.at[pl.ds(0, EPW)])
    pltpu.sync_copy(dst_hbm.at[pl.ds(base, EPW)], dst_v.at[pl.ds(0, EPW)])
    pltpu.sync_copy(et_hbm.at[pl.ds(base, EPW)], et_v.at[pl.ds(0, EPW)])

    def body(j, _):
        s16 = src_v[pl.ds(j * L, L)]
        d16 = dst_v[pl.ds(j * L, L)]
        t16 = et_v[pl.ds(j * L, L)]
        seg = jnp.clip(d16 * R + t16, 0, NRP - 1)
        w_v[pl.ds(j * L, L)] = plsc.load_gather(inv_v, [seg])
        gi_v[pl.ds(j * L, L)] = jnp.clip(t16 * N + s16, 0, NR - 1)
        return 0

    lax.fori_loop(0, (EPW + L - 1) // L, body, 0)
    pltpu.sync_copy(w_v.at[pl.ds(0, EPW)], w_hbm.at[pl.ds(base, EPW)])
    pltpu.sync_copy(gi_v.at[pl.ds(0, EPW)], gidx_hbm.at[pl.ds(base, EPW)])


@functools.partial(
    pl.kernel,
    out_type=jax.ShapeDtypeStruct((NC, N, 128), jnp.float32),
    scratch_types=[
        pltpu.VMEM((8 * KA,), jnp.int32),
        pltpu.VMEM((8 * KA,), jnp.float32),
        pltpu.VMEM((8, KA), jnp.int32),
        [pltpu.VMEM((KA, 128), jnp.float32) for _ in range(3)],
        [pltpu.SemaphoreType.DMA for _ in range(3)],
        [pltpu.SemaphoreType.DMA for _ in range(3)],
        pltpu.VMEM((40, 128), jnp.float32),
        pltpu.VMEM_SHARED((N, 128), jnp.float32),
    ],
    **_MESH,
)
def _agg_k(tlo_hbm, thi_hbm, gidx_hbm, dst3_hbm, w_hbm, out_hbm,
           gi_v, w_v, ds3_v, rows, gsems, ssems, zbuf, acc_sh):
    c = lax.axis_index("c")
    s = lax.axis_index("s")
    base = s * EPT

    zero = jnp.zeros((L,), jnp.float32)

    def zb(j, _):
        for u in range(8):
            zbuf[j, pl.ds(u * L, L)] = zero
        return 0

    lax.fori_loop(0, 40, zb, 0)
    for kk in range(16):
        ch = s + NS * kk

        @pl.when(ch < N // 40)
        def _(ch=ch):
            pltpu.sync_copy(zbuf, acc_sh.at[pl.ds(ch * 40, 40)])

    plsc.subcore_barrier()

    def scale_chunk(bb, loff):
        def sc(i, _, _bb=bb, _loff=loff):
            w16 = w_v[pl.ds(_loff + i * L, L)]
            for u in range(L):
                wj = w16[u]
                ri = i * L + u
                for g in range(8):
                    rows[_bb][ri, pl.ds(g * L, L)] = (
                        rows[_bb][ri, pl.ds(g * L, L)] * wj)
            return 0

        lax.fori_loop(0, KA // L, sc, 0)

    def run(table):
        def group(g, _):
            pltpu.sync_copy(gidx_hbm.at[pl.ds(base + g * 8 * KA, 8 * KA)], gi_v)
            pltpu.sync_copy(w_hbm.at[pl.ds(base + g * 8 * KA, 8 * KA)], w_v)
            pltpu.sync_copy(dst3_hbm.at[s, pl.ds(g * 8, 8)], ds3_v)
            gd = [None, None, None]
            sall = [None] * 8

            def finish(b):
                bb = b % 3
                gd[bb].wait()
                scale_chunk(bb, b * KA)
                sall[b] = pltpu.async_copy(rows[bb], acc_sh.at[ds3_v.at[b]],
                                           ssems[bb], add=True)

            for b in range(8):
                bb = b % 3
                act = g * 8 + b < NCH_A

                @pl.when(act)
                def _(b=b, bb=bb):
                    if b >= 3 and sall[b - 3] is not None:
                        sall[b - 3].wait()
                    gd[bb] = pltpu.async_copy(
                        table.at[gi_v.at[pl.ds(b * KA, KA)]], rows[bb], gsems[bb])

                if b >= 2:
                    @pl.when(g * 8 + b - 2 < NCH_A)
                    def _(b=b):
                        finish(b - 2)

            for b in (6, 7):
                @pl.when(g * 8 + b < NCH_A)
                def _(b=b):
                    finish(b)

            for b in range(8):
                if sall[b] is not None:
                    cond = (g * 8 + b < NCH_A)
                    if b + 3 < 8:
                        cond = cond & (g * 8 + b + 3 >= NCH_A)

                    @pl.when(cond)
                    def _(b=b):
                        sall[b].wait()
            return 0

        lax.fori_loop(0, NGRP, group, 0)

    @pl.when(c == 0)
    def _():
        run(tlo_hbm)

    @pl.when(c == 1)
    def _():
        run(thi_hbm)

    plsc.subcore_barrier()

    for kk in range(8):
        ch = s + NS * kk

        @pl.when(ch < 125)
        def _(ch=ch):
            pltpu.sync_copy(acc_sh.at[pl.ds(ch * 80, 80)],
                            out_hbm.at[c, pl.ds(ch * 80, 80)])


@functools.partial(
    pl.kernel,
    out_type=jax.ShapeDtypeStruct((NW, L), jnp.float32),
    scratch_types=[
        pltpu.VMEM((EPAD,), jnp.int32),
        pltpu.VMEM((EPAD,), jnp.int32),
        pltpu.VMEM((EPAD,), jnp.float32),
        pltpu.VMEM((EPAD,), jnp.float32),
        pltpu.VMEM((D // 2,), jnp.int32),
        [pltpu.VMEM((KT, D // 2), jnp.int32) for _ in range(2)],
        [pltpu.VMEM((KT, D // 2), jnp.int32) for _ in range(2)],
        [pltpu.SemaphoreType.DMA for _ in range(4)],
        pltpu.VMEM((D,), jnp.float32),
        pltpu.VMEM((L,), jnp.float32),
        pltpu.VMEM((L,), jnp.float32),
    ],
    **_MESH,
)
def _att_k(u_hbm, v_hbm, src_hbm, dst_hbm, conf_hbm, w2_hbm, b2_hbm, out_hbm,
           src_v, dst_v, conf_v, pd_v, w2_v, urows, vrows, sems, tr_v, ps_v, b2_v):
    wid = _wid()
    base = wid * EPW
    zero = jnp.zeros((L,), jnp.float32)
    zeroi = jnp.zeros((L,), jnp.int32)
    for t in range(EPW // L, EPAD // L):
        src_v[pl.ds(t * L, L)] = zeroi
        dst_v[pl.ds(t * L, L)] = zeroi
    pltpu.sync_copy(src_hbm.at[pl.ds(base, EPW)], src_v.at[pl.ds(0, EPW)])
    pltpu.sync_copy(dst_hbm.at[pl.ds(base, EPW)], dst_v.at[pl.ds(0, EPW)])
    pltpu.sync_copy(conf_hbm.at[pl.ds(base, EPW)], conf_v.at[pl.ds(0, EPW)])
    pltpu.sync_copy(w2_hbm, w2_v)
    pltpu.sync_copy(b2_hbm, b2_v)
    lane16 = lax.iota(jnp.int32, L)
    bzero = jnp.zeros((2 * L,), jnp.bfloat16)
    w2g = [plsc.bitcast(w2_v[pl.ds(g * L, L)], jnp.bfloat16)
           for g in range(D // (2 * L))]

    def do_chunk(i, b):
        for sub in range(KT // L):
            for u in range(L):
                ri = sub * L + u
                acc0 = bzero
                acc1 = bzero
                for g in range(D // (2 * L)):
                    uu = plsc.bitcast(urows[b][ri, pl.ds(g * L, L)], jnp.bfloat16)
                    vv = plsc.bitcast(vrows[b][ri, pl.ds(g * L, L)], jnp.bfloat16)
                    z = jnp.maximum(uu + vv, bzero)
                    if g % 2 == 0:
                        acc0 = acc0 + z * w2g[g]
                    else:
                        acc1 = acc1 + z * w2g[g]
                fa, fb = plsc.unpack(acc0 + acc1, format=plsc.PackFormat.INTERLEAVED,
                                     preferred_element_type=jnp.float32)
                tr_v[pl.ds(u * L, L)] = fa + fb
            pd16 = jnp.zeros((L,), jnp.float32)
            for lpos in range(L):
                pd16 = pd16 + plsc.load_gather(tr_v, [lane16 * L + lpos])
            pd_v[pl.ds(i * KT + sub * L, L)] = pd16

    def issue(i, b):
        d0 = pltpu.async_copy(u_hbm.at[src_v.at[pl.ds(i * KT, KT)]],
                              urows[b], sems[2 * b])
        d1 = pltpu.async_copy(v_hbm.at[dst_v.at[pl.ds(i * KT, KT)]],
                              vrows[b], sems[2 * b + 1])
        return d0, d1

    def pair(p, _):
        i0 = 2 * p
        da = issue(i0, 0)
        db = issue(i0 + 1, 1)
        da[0].wait(); da[1].wait()
        do_chunk(i0, 0)
        db[0].wait(); db[1].wait()
        do_chunk(i0 + 1, 1)
        return 0

    lax.fori_loop(0, (NCH_T - 1) // 2, pair, 0)
    dl = issue(NCH_T - 1, 0)
    dl[0].wait(); dl[1].wait()
    do_chunk(NCH_T - 1, 0)

    nfull = EPW // L
    b2 = b2_v[...]

    def sig(j, acc):
        pd16 = pd_v[pl.ds(j * L, L)] + b2
        cf16 = conf_v[pl.ds(j * L, L)]
        return acc + cf16 / (1.0 + jnp.exp(-pd16))

    asum = lax.fori_loop(0, nfull, sig, jnp.zeros((L,), jnp.float32))
    pd16 = pd_v[pl.ds(nfull * L, L)] + b2
    cf16 = conf_v[pl.ds(nfull * L, L)]
    tailv = cf16 / (1.0 + jnp.exp(-pd16))
    asum = asum + jnp.where(lane16 < (EPW - nfull * L), tailv, zero)
    ps_v[...] = asum
    pltpu.sync_copy(ps_v, out_hbm.at[wid])


def _transform_body(h_ref, rel_ref, root_ref, bias_ref, tlo_ref, thi_ref, hroot_ref):
    r = pl.program_id(1)
    hb = h_ref[...]
    t = jnp.dot(hb.astype(jnp.bfloat16), rel_ref[0].astype(jnp.bfloat16),
                preferred_element_type=jnp.float32)
    tlo_ref[...] = t[:, :128]
    thi_ref[...] = t[:, 128:]

    @pl.when(r == 0)
    def _():
        hroot_ref[...] = (jnp.dot(hb, root_ref[...], preferred_element_type=jnp.float32)
                          + bias_ref[...])


def _transform(h, rel, root, bias2d):
    return pl.pallas_call(
        _transform_body,
        grid=(NB, R),
        in_specs=[
            pl.BlockSpec((BN, D), lambda i, r: (i, 0)),
            pl.BlockSpec((1, D, D), lambda i, r: (r, 0, 0)),
            pl.BlockSpec((D, D), lambda i, r: (0, 0)),
            pl.BlockSpec((1, D), lambda i, r: (0, 0)),
        ],
        out_specs=[
            pl.BlockSpec((BN, 128), lambda i, r: (r * NB + i, 0)),
            pl.BlockSpec((BN, 128), lambda i, r: (r * NB + i, 0)),
            pl.BlockSpec((BN, D), lambda i, r: (i, 0)),
        ],
        out_shape=[
            jax.ShapeDtypeStruct((NR, 128), jnp.float32),
            jax.ShapeDtypeStruct((NR, 128), jnp.float32),
            jax.ShapeDtypeStruct((N, D), jnp.float32),
        ],
    )(h, rel, root, bias2d)


def _combine_body(hroot_ref, a0_ref, a1_ref, o_ref):
    acc = jnp.concatenate([a0_ref[0], a1_ref[0]], axis=-1)
    o_ref[...] = jnp.maximum(hroot_ref[...] + acc, 0.0)


def _combine(hroot, acc):
    return pl.pallas_call(
        _combine_body,
        grid=(NB,),
        in_specs=[
            pl.BlockSpec((BN, D), lambda i: (i, 0)),
            pl.BlockSpec((1, BN, 128), lambda i: (0, i, 0)),
            pl.BlockSpec((1, BN, 128), lambda i: (1, i, 0)),
        ],
        out_specs=pl.BlockSpec((BN, D), lambda i: (i, 0)),
        out_shape=jax.ShapeDtypeStruct((N, D), jnp.float32),
    )(hroot, acc, acc)


def _combine_uv_body(hroot_ref, a0_ref, a1_ref, w1a_ref, w1b_ref, b1_ref,
                     h_ref, u_ref, v_ref):
    acc = jnp.concatenate([a0_ref[0], a1_ref[0]], axis=-1)
    hf = hroot_ref[...] + acc
    h_ref[...] = hf
    u_ref[...] = (jnp.dot(hf, w1a_ref[...], preferred_element_type=jnp.float32)
                  + b1_ref[...]).astype(jnp.bfloat16)
    v_ref[...] = jnp.dot(hf, w1b_ref[...],
                         preferred_element_type=jnp.float32).astype(jnp.bfloat16)


def _combine_uv(hroot, acc, w1a, w1b, b1_2d):
    return pl.pallas_call(
        _combine_uv_body,
        grid=(NB,),
        in_specs=[
            pl.BlockSpec((BN, D), lambda i: (i, 0)),
            pl.BlockSpec((1, BN, 128), lambda i: (0, i, 0)),
            pl.BlockSpec((1, BN, 128), lambda i: (1, i, 0)),
            pl.BlockSpec((D, D), lambda i: (0, 0)),
            pl.BlockSpec((D, D), lambda i: (0, 0)),
            pl.BlockSpec((1, D), lambda i: (0, 0)),
        ],
        out_specs=[
            pl.BlockSpec((BN, D), lambda i: (i, 0)),
            pl.BlockSpec((BN, D), lambda i: (i, 0)),
            pl.BlockSpec((BN, D), lambda i: (i, 0)),
        ],
        out_shape=[
            jax.ShapeDtypeStruct((N, D), jnp.float32),
            jax.ShapeDtypeStruct((N, D), jnp.bfloat16),
            jax.ShapeDtypeStruct((N, D), jnp.bfloat16),
        ],
    )(hroot, acc, acc, w1a, w1b, b1_2d)


def _final_body(h_ref, p_ref, bw_ref, bb_ref, mw_ref, mb_ref, vw_ref, vb_ref,
                pm_ref, pv_ref):
    s = jnp.sum(p_ref[...]) * (1.0 / E)
    b = (jnp.dot(h_ref[...] * s, bw_ref[...], preferred_element_type=jnp.float32)
         + bb_ref[...])
    pm_ref[...] = jnp.dot(b, mw_ref[...], preferred_element_type=jnp.float32) + mb_ref[...]
    vv = jnp.dot(b, vw_ref[...], preferred_element_type=jnp.float32) + vb_ref[...]
    pv_ref[...] = jnp.log1p(jnp.exp(-jnp.abs(vv))) + jnp.maximum(vv, 0.0)


def _final(h, part, bw, bb2d, mw, mb2d, vw, vb2d):
    wspec = pl.BlockSpec((D, D), lambda i: (0, 0))
    bspec = pl.BlockSpec((1, D), lambda i: (0, 0))
    return pl.pallas_call(
        _final_body,
        grid=(NB,),
        in_specs=[
            pl.BlockSpec((BN, D), lambda i: (i, 0)),
            pl.BlockSpec((4, 128), lambda i: (0, 0)),
            wspec, bspec, wspec, bspec, wspec, bspec,
        ],
        out_specs=[
            pl.BlockSpec((BN, D), lambda i: (i, 0)),
            pl.BlockSpec((BN, D), lambda i: (i, 0)),
        ],
        out_shape=[
            jax.ShapeDtypeStruct((N, D), jnp.float32),
            jax.ShapeDtypeStruct((N, D), jnp.float32),
        ],
    )(h, part, bw, bb2d, mw, mb2d, vw, vb2d)


def kernel(x, edge_index, edge_type, edge_confidence,
           root0, rel0, bias0, root1, rel1, bias1, root2, rel2, bias2,
           att_w1, att_b1, att_w2, att_b2,
           bayes_w, bayes_b, mean_w, mean_b, var_w, var_b):
    src = edge_index[0]
    dst = edge_index[1]
    et = edge_type

    hist = _hist_k(dst, et)
    inv = _inv(hist.reshape(NW, 640, 128))
    w_e, gidx = _wg_k(src, dst, et, inv.reshape(NRP))
    pad = NGRP * 8 * KA - EPT
    w_p = jnp.pad(w_e, (0, pad))
    gidx_p = jnp.pad(gidx, (0, pad))
    dst3 = jnp.pad(dst.reshape(NS, NCH_A, KA), ((0, 0), (0, NGRP * 8 - NCH_A), (0, 0)))

    h = x
    layers = [(root0, rel0, bias0), (root1, rel1, bias1), (root2, rel2, bias2)]
    u = v = None
    for li, (root, rel, bias) in enumerate(layers):
        tlo, thi, hroot = _transform(h, rel, root, bias.reshape(1, D))
        acc = _agg_k(tlo, thi, gidx_p, dst3, w_p)
        if li < 2:
            h = _combine(hroot, acc)
        else:
            h, u, v = _combine_uv(hroot, acc, att_w1[:D], att_w1[D:],
                                  att_b1.reshape(1, D))

    b2v = jnp.broadcast_to(att_b2.reshape(1), (L,)).astype(jnp.float32)
    u32 = lax.bitcast_convert_type(u.reshape(N, D // 2, 2), jnp.int32)
    v32 = lax.bitcast_convert_type(v.reshape(N, D // 2, 2), jnp.int32)
    w2p = lax.bitcast_convert_type(
        att_w2.reshape(D).astype(jnp.bfloat16).reshape(D // 2, 2), jnp.int32)
    part = _att_k(u32, v32, src, dst, edge_confidence, w2p, b2v)
    pm, pv = _final(h, part.reshape(4, 128),
                    bayes_w, bayes_b.reshape(1, D),
                    mean_w, mean_b.reshape(1, D),
                    var_w, var_b.reshape(1, D))
    return pm, pv


# speedup vs baseline: 1.0012x; 1.0012x over previous
import functools

import jax
import jax.numpy as jnp
from jax import lax
from jax.experimental import pallas as pl
from jax.experimental.pallas import tpu as pltpu
from jax.experimental.pallas import tpu_sc as plsc

N = 10000
E = 160000
R = 8
D = 256
NR = N * R
NRP = 81920
NC, NS, L = 2, 16, 16
NW = NC * NS
EPW = E // NW
EPT = E // NS
KA = 80
NCH_A = EPT // KA
NGRP = (NCH_A + 7) // 8
KT = 64
NCH_T = (EPW + KT - 1) // KT
EPAD = NCH_T * KT
NB = 5
BN = N // NB

_MESH = dict(mesh=plsc.VectorSubcoreMesh(core_axis_name="c", subcore_axis_name="s",
                                         num_cores=NC, num_subcores=NS),
             compiler_params=pltpu.CompilerParams(needs_layout_passes=False))


def _wid():
    return lax.axis_index("s") * NC + lax.axis_index("c")


@functools.partial(
    pl.kernel,
    out_type=jax.ShapeDtypeStruct((NW, NRP), jnp.float32),
    scratch_types=[
        pltpu.VMEM((NRP,), jnp.float32),
        pltpu.VMEM((EPW + 16,), jnp.int32),
        pltpu.VMEM((EPW + 16,), jnp.int32),
    ],
    **_MESH,
)
def _hist_k(dst_hbm, et_hbm, out_hbm, hist_v, dst_v, et_v):
    wid = _wid()
    base = wid * EPW
    pltpu.sync_copy(dst_hbm.at[pl.ds(base, EPW)], dst_v.at[pl.ds(0, EPW)])
    pltpu.sync_copy(et_hbm.at[pl.ds(base, EPW)], et_v.at[pl.ds(0, EPW)])
    zero = jnp.zeros((L,), jnp.float32)

    def zbody(j, _):
        for u in range(16):
            hist_v[pl.ds((j * 16 + u) * L, L)] = zero
        return 0

    lax.fori_loop(0, NRP // (16 * L), zbody, 0)
    ones = jnp.ones((L,), jnp.float32)
    nfull = EPW // L

    def body(j, _):
        d16 = dst_v[pl.ds(j * L, L)]
        t16 = et_v[pl.ds(j * L, L)]
        seg = jnp.clip(d16 * R + t16, 0, NRP - 1)
        plsc.addupdate_scatter(hist_v, [seg], ones)
        return 0

    lax.fori_loop(0, nfull, body, 0)
    d16 = dst_v[pl.ds(nfull * L, L)]
    t16 = et_v[pl.ds(nfull * L, L)]
    seg = jnp.clip(d16 * R + t16, 0, NRP - 1)
    lane = lax.iota(jnp.int32, L)
    plsc.addupdate_scatter(hist_v, [seg], ones, mask=lane < (EPW - nfull * L))
    pltpu.sync_copy(hist_v, out_hbm.at[wid])


def _inv_body(h_ref, o_ref):
    cnt = jnp.sum(h_ref[...], axis=0)
    o_ref[...] = 1.0 / jnp.maximum(cnt, 1.0)


def _inv(hist3):
    return pl.pallas_call(
        _inv_body,
        grid=(10,),
        in_specs=[pl.BlockSpec((NW, 64, 128), lambda i: (0, i, 0))],
        out_specs=pl.BlockSpec((64, 128), lambda i: (i, 0)),
        out_shape=jax.ShapeDtypeStruct((640, 128), jnp.float32),
    )(hist3)


@functools.partial(
    pl.kernel,
    out_type=(
        jax.ShapeDtypeStruct((E,), jnp.float32),
        jax.ShapeDtypeStruct((E,), jnp.int32),
    ),
    scratch_types=[
        pltpu.VMEM((NRP,), jnp.float32),
        pltpu.VMEM((EPW + 16,), jnp.int32),
        pltpu.VMEM((EPW + 16,), jnp.int32),
        pltpu.VMEM((EPW + 16,), jnp.int32),
        pltpu.VMEM((EPW + 16,), jnp.float32),
        pltpu.VMEM((EPW + 16,), jnp.int32),
    ],
    **_MESH,
)
def _wg_k(src_hbm, dst_hbm, et_hbm, inv_hbm, w_hbm, gidx_hbm,
          inv_v, src_v, dst_v, et_v, w_v, gi_v):
    wid = _wid()
    base = wid * EPW
    pltpu.sync_copy(inv_hbm, inv_v)
    pltpu.sync_copy(src_hbm.at[pl.ds(base, EPW)], src_v.at[pl.ds(0, EPW)])
    pltpu.sync_copy(dst_hbm.at[pl.ds(base, EPW)], dst_v.at[pl.ds(0, EPW)])
    pltpu.sync_copy(et_hbm.at[pl.ds(base, EPW)], et_v.at[pl.ds(0, EPW)])

    def body(j, _):
        s16 = src_v[pl.ds(j * L, L)]
        d16 = dst_v[pl.ds(j * L, L)]
        t16 = et_v[pl.ds(j * L, L)]
        seg = jnp.clip(d16 * R + t16, 0, NRP - 1)
        w_v[pl.ds(j * L, L)] = plsc.load_gather(inv_v, [seg])
        gi_v[pl.ds(j * L, L)] = jnp.clip(t16 * N + s16, 0, NR - 1)
        return 0

    lax.fori_loop(0, (EPW + L - 1) // L, body, 0)
    pltpu.sync_copy(w_v.at[pl.ds(0, EPW)], w_hbm.at[pl.ds(base, EPW)])
    pltpu.sync_copy(gi_v.at[pl.ds(0, EPW)], gidx_hbm.at[pl.ds(base, EPW)])


@functools.partial(
    pl.kernel,
    out_type=jax.ShapeDtypeStruct((NC, N, 128), jnp.float32),
    scratch_types=[
        pltpu.VMEM((8 * KA,), jnp.int32),
        pltpu.VMEM((8 * KA,), jnp.float32),
        pltpu.VMEM((8, KA), jnp.int32),
        [pltpu.VMEM((KA, 128), jnp.float32) for _ in range(3)],
        [pltpu.SemaphoreType.DMA for _ in range(3)],
        [pltpu.SemaphoreType.DMA for _ in range(3)],
        pltpu.VMEM((40, 128), jnp.float32),
        pltpu.VMEM_SHARED((N, 128), jnp.float32),
    ],
    **_MESH,
)
def _agg_k(tlo_hbm, thi_hbm, gidx_hbm, dst3_hbm, w_hbm, out_hbm,
           gi_v, w_v, ds3_v, rows, gsems, ssems, zbuf, acc_sh):
    c = lax.axis_index("c")
    s = lax.axis_index("s")
    base = s * EPT

    zero = jnp.zeros((L,), jnp.float32)

    def zb(j, _):
        for u in range(8):
            zbuf[j, pl.ds(u * L, L)] = zero
        return 0

    lax.fori_loop(0, 40, zb, 0)
    for kk in range(16):
        ch = s + NS * kk

        @pl.when(ch < N // 40)
        def _(ch=ch):
            pltpu.sync_copy(zbuf, acc_sh.at[pl.ds(ch * 40, 40)])

    plsc.subcore_barrier()

    def scale_chunk(bb, loff):
        def sc(i, _, _bb=bb, _loff=loff):
            w16 = w_v[pl.ds(_loff + i * L, L)]
            for u in range(L):
                wj = w16[u]
                ri = i * L + u
                for g in range(8):
                    rows[_bb][ri, pl.ds(g * L, L)] = (
                        rows[_bb][ri, pl.ds(g * L, L)] * wj)
            return 0

        lax.fori_loop(0, KA // L, sc, 0)

    def run(table):
        def group(g, _):
            pltpu.sync_copy(gidx_hbm.at[pl.ds(base + g * 8 * KA, 8 * KA)], gi_v)
            pltpu.sync_copy(w_hbm.at[pl.ds(base + g * 8 * KA, 8 * KA)], w_v)
            pltpu.sync_copy(dst3_hbm.at[s, pl.ds(g * 8, 8)], ds3_v)
            gd = [None, None, None]
            sall = [None] * 8

            def finish(b):
                bb = b % 3
                gd[bb].wait()
                scale_chunk(bb, b * KA)
                sall[b] = pltpu.async_copy(rows[bb], acc_sh.at[ds3_v.at[b]],
                                           ssems[bb], add=True)

            for b in range(8):
                bb = b % 3
                act = g * 8 + b < NCH_A

                @pl.when(act)
                def _(b=b, bb=bb):
                    if b >= 3 and sall[b - 3] is not None:
                        sall[b - 3].wait()
                    gd[bb] = pltpu.async_copy(
                        table.at[gi_v.at[pl.ds(b * KA, KA)]], rows[bb], gsems[bb])

                if b >= 2:
                    @pl.when(g * 8 + b - 2 < NCH_A)
                    def _(b=b):
                        finish(b - 2)

            for b in (6, 7):
                @pl.when(g * 8 + b < NCH_A)
                def _(b=b):
                    finish(b)

            for b in range(8):
                if sall[b] is not None:
                    cond = (g * 8 + b < NCH_A)
                    if b + 3 < 8:
                        cond = cond & (g * 8 + b + 3 >= NCH_A)

                    @pl.when(cond)
                    def _(b=b):
                        sall[b].wait()
            return 0

        lax.fori_loop(0, NGRP, group, 0)

    @pl.when(c == 0)
    def _():
        run(tlo_hbm)

    @pl.when(c == 1)
    def _():
        run(thi_hbm)

    plsc.subcore_barrier()

    for kk in range(8):
        ch = s + NS * kk

        @pl.when(ch < 125)
        def _(ch=ch):
            pltpu.sync_copy(acc_sh.at[pl.ds(ch * 80, 80)],
                            out_hbm.at[c, pl.ds(ch * 80, 80)])


@functools.partial(
    pl.kernel,
    out_type=jax.ShapeDtypeStruct((NW, L), jnp.float32),
    scratch_types=[
        pltpu.VMEM((EPAD,), jnp.int32),
        pltpu.VMEM((EPAD,), jnp.int32),
        pltpu.VMEM((EPAD,), jnp.float32),
        pltpu.VMEM((EPAD,), jnp.float32),
        pltpu.VMEM((D // 2,), jnp.int32),
        [pltpu.VMEM((KT, D // 2), jnp.int32) for _ in range(2)],
        [pltpu.VMEM((KT, D // 2), jnp.int32) for _ in range(2)],
        [pltpu.SemaphoreType.DMA for _ in range(4)],
        pltpu.VMEM((D,), jnp.float32),
        pltpu.VMEM((L,), jnp.float32),
        pltpu.VMEM((L,), jnp.float32),
    ],
    **_MESH,
)
def _att_k(u_hbm, v_hbm, src_hbm, dst_hbm, conf_hbm, w2_hbm, b2_hbm, out_hbm,
           src_v, dst_v, conf_v, pd_v, w2_v, urows, vrows, sems, tr_v, ps_v, b2_v):
    wid = _wid()
    base = wid * EPW
    zero = jnp.zeros((L,), jnp.float32)
    zeroi = jnp.zeros((L,), jnp.int32)
    for t in range(EPW // L, EPAD // L):
        src_v[pl.ds(t * L, L)] = zeroi
        dst_v[pl.ds(t * L, L)] = zeroi
    pltpu.sync_copy(src_hbm.at[pl.ds(base, EPW)], src_v.at[pl.ds(0, EPW)])
    pltpu.sync_copy(dst_hbm.at[pl.ds(base, EPW)], dst_v.at[pl.ds(0, EPW)])
    pltpu.sync_copy(conf_hbm.at[pl.ds(base, EPW)], conf_v.at[pl.ds(0, EPW)])
    pltpu.sync_copy(w2_hbm, w2_v)
    pltpu.sync_copy(b2_hbm, b2_v)
    lane16 = lax.iota(jnp.int32, L)
    bzero = jnp.zeros((2 * L,), jnp.bfloat16)
    w2g = [plsc.bitcast(w2_v[pl.ds(g * L, L)], jnp.bfloat16)
           for g in range(D // (2 * L))]

    def do_chunk(i, b):
        for sub in range(KT // L):
            for u in range(L):
                ri = sub * L + u
                acc0 = bzero
                acc1 = bzero
                for g in range(D // (2 * L)):
                    uu = plsc.bitcast(urows[b][ri, pl.ds(g * L, L)], jnp.bfloat16)
                    vv = plsc.bitcast(vrows[b][ri, pl.ds(g * L, L)], jnp.bfloat16)
                    z = jnp.maximum(uu + vv, bzero)
                    if g % 2 == 0:
                        acc0 = acc0 + z * w2g[g]
                    else:
                        acc1 = acc1 + z * w2g[g]
                fa, fb = plsc.unpack(acc0 + acc1, format=plsc.PackFormat.INTERLEAVED,
                                     preferred_element_type=jnp.float32)
                tr_v[pl.ds(u * L, L)] = fa + fb
            pd16 = jnp.zeros((L,), jnp.float32)
            for lpos in range(L):
                pd16 = pd16 + plsc.load_gather(tr_v, [lane16 * L + lpos])
            pd_v[pl.ds(i * KT + sub * L, L)] = pd16

    def issue(i, b):
        d0 = pltpu.async_copy(u_hbm.at[src_v.at[pl.ds(i * KT, KT)]],
                              urows[b], sems[2 * b])
        d1 = pltpu.async_copy(v_hbm.at[dst_v.at[pl.ds(i * KT, KT)]],
                              vrows[b], sems[2 * b + 1])
        return d0, d1

    def pair(p, _):
        i0 = 2 * p
        da = issue(i0, 0)
        db = issue(i0 + 1, 1)
        da[0].wait(); da[1].wait()
        do_chunk(i0, 0)
        db[0].wait(); db[1].wait()
        do_chunk(i0 + 1, 1)
        return 0

    lax.fori_loop(0, (NCH_T - 1) // 2, pair, 0)
    dl = issue(NCH_T - 1, 0)
    dl[0].wait(); dl[1].wait()
    do_chunk(NCH_T - 1, 0)

    nfull = EPW // L
    b2 = b2_v[...]

    def sig(j, acc):
        pd16 = pd_v[pl.ds(j * L, L)] + b2
        cf16 = conf_v[pl.ds(j * L, L)]
        return acc + cf16 / (1.0 + jnp.exp(-pd16))

    asum = lax.fori_loop(0, nfull, sig, jnp.zeros((L,), jnp.float32))
    pd16 = pd_v[pl.ds(nfull * L, L)] + b2
    cf16 = conf_v[pl.ds(nfull * L, L)]
    tailv = cf16 / (1.0 + jnp.exp(-pd16))
    asum = asum + jnp.where(lane16 < (EPW - nfull * L), tailv, zero)
    ps_v[...] = asum
    pltpu.sync_copy(ps_v, out_hbm.at[wid])


def _transform_body(h_ref, rel_ref, root_ref, bias_ref, tlo_ref, thi_ref, hroot_ref):
    r = pl.program_id(1)
    hb = h_ref[...]
    t = jnp.dot(hb, rel_ref[0], preferred_element_type=jnp.float32)
    tlo_ref[...] = t[:, :128]
    thi_ref[...] = t[:, 128:]

    @pl.when(r == 0)
    def _():
        hroot_ref[...] = (jnp.dot(hb, root_ref[...], preferred_element_type=jnp.float32)
                          + bias_ref[...])


def _transform(h, rel, root, bias2d):
    return pl.pallas_call(
        _transform_body,
        grid=(NB, R),
        in_specs=[
            pl.BlockSpec((BN, D), lambda i, r: (i, 0)),
            pl.BlockSpec((1, D, D), lambda i, r: (r, 0, 0)),
            pl.BlockSpec((D, D), lambda i, r: (0, 0)),
            pl.BlockSpec((1, D), lambda i, r: (0, 0)),
        ],
        out_specs=[
            pl.BlockSpec((BN, 128), lambda i, r: (r * NB + i, 0)),
            pl.BlockSpec((BN, 128), lambda i, r: (r * NB + i, 0)),
            pl.BlockSpec((BN, D), lambda i, r: (i, 0)),
        ],
        out_shape=[
            jax.ShapeDtypeStruct((NR, 128), jnp.float32),
            jax.ShapeDtypeStruct((NR, 128), jnp.float32),
            jax.ShapeDtypeStruct((N, D), jnp.float32),
        ],
    )(h, rel, root, bias2d)


def _combine_body(hroot_ref, a0_ref, a1_ref, o_ref):
    acc = jnp.concatenate([a0_ref[0], a1_ref[0]], axis=-1)
    o_ref[...] = jnp.maximum(hroot_ref[...] + acc, 0.0)


def _combine(hroot, acc):
    return pl.pallas_call(
        _combine_body,
        grid=(NB,),
        in_specs=[
            pl.BlockSpec((BN, D), lambda i: (i, 0)),
            pl.BlockSpec((1, BN, 128), lambda i: (0, i, 0)),
            pl.BlockSpec((1, BN, 128), lambda i: (1, i, 0)),
        ],
        out_specs=pl.BlockSpec((BN, D), lambda i: (i, 0)),
        out_shape=jax.ShapeDtypeStruct((N, D), jnp.float32),
    )(hroot, acc, acc)


def _combine_uv_body(hroot_ref, a0_ref, a1_ref, w1a_ref, w1b_ref, b1_ref,
                     h_ref, u_ref, v_ref):
    acc = jnp.concatenate([a0_ref[0], a1_ref[0]], axis=-1)
    hf = hroot_ref[...] + acc
    h_ref[...] = hf
    u_ref[...] = (jnp.dot(hf, w1a_ref[...], preferred_element_type=jnp.float32)
                  + b1_ref[...]).astype(jnp.bfloat16)
    v_ref[...] = jnp.dot(hf, w1b_ref[...],
                         preferred_element_type=jnp.float32).astype(jnp.bfloat16)


def _combine_uv(hroot, acc, w1a, w1b, b1_2d):
    return pl.pallas_call(
        _combine_uv_body,
        grid=(NB,),
        in_specs=[
            pl.BlockSpec((BN, D), lambda i: (i, 0)),
            pl.BlockSpec((1, BN, 128), lambda i: (0, i, 0)),
            pl.BlockSpec((1, BN, 128), lambda i: (1, i, 0)),
            pl.BlockSpec((D, D), lambda i: (0, 0)),
            pl.BlockSpec((D, D), lambda i: (0, 0)),
            pl.BlockSpec((1, D), lambda i: (0, 0)),
        ],
        out_specs=[
            pl.BlockSpec((BN, D), lambda i: (i, 0)),
            pl.BlockSpec((BN, D), lambda i: (i, 0)),
            pl.BlockSpec((BN, D), lambda i: (i, 0)),
        ],
        out_shape=[
            jax.ShapeDtypeStruct((N, D), jnp.float32),
            jax.ShapeDtypeStruct((N, D), jnp.bfloat16),
            jax.ShapeDtypeStruct((N, D), jnp.bfloat16),
        ],
    )(hroot, acc, acc, w1a, w1b, b1_2d)


def _final_body(h_ref, p_ref, bw_ref, bb_ref, mw_ref, mb_ref, vw_ref, vb_ref,
                pm_ref, pv_ref):
    s = jnp.sum(p_ref[...]) * (1.0 / E)
    b = (jnp.dot(h_ref[...] * s, bw_ref[...], preferred_element_type=jnp.float32)
         + bb_ref[...])
    pm_ref[...] = jnp.dot(b, mw_ref[...], preferred_element_type=jnp.float32) + mb_ref[...]
    vv = jnp.dot(b, vw_ref[...], preferred_element_type=jnp.float32) + vb_ref[...]
    pv_ref[...] = jnp.log1p(jnp.exp(-jnp.abs(vv))) + jnp.maximum(vv, 0.0)


def _final(h, part, bw, bb2d, mw, mb2d, vw, vb2d):
    wspec = pl.BlockSpec((D, D), lambda i: (0, 0))
    bspec = pl.BlockSpec((1, D), lambda i: (0, 0))
    return pl.pallas_call(
        _final_body,
        grid=(NB,),
        in_specs=[
            pl.BlockSpec((BN, D), lambda i: (i, 0)),
            pl.BlockSpec((4, 128), lambda i: (0, 0)),
            wspec, bspec, wspec, bspec, wspec, bspec,
        ],
        out_specs=[
            pl.BlockSpec((BN, D), lambda i: (i, 0)),
            pl.BlockSpec((BN, D), lambda i: (i, 0)),
        ],
        out_shape=[
            jax.ShapeDtypeStruct((N, D), jnp.float32),
            jax.ShapeDtypeStruct((N, D), jnp.float32),
        ],
    )(h, part, bw, bb2d, mw, mb2d, vw, vb2d)


def kernel(x, edge_index, edge_type, edge_confidence,
           root0, rel0, bias0, root1, rel1, bias1, root2, rel2, bias2,
           att_w1, att_b1, att_w2, att_b2,
           bayes_w, bayes_b, mean_w, mean_b, var_w, var_b):
    src = edge_index[0]
    dst = edge_index[1]
    et = edge_type

    hist = _hist_k(dst, et)
    inv = _inv(hist.reshape(NW, 640, 128))
    w_e, gidx = _wg_k(src, dst, et, inv.reshape(NRP))
    pad = NGRP * 8 * KA - EPT
    w_p = jnp.pad(w_e, (0, pad))
    gidx_p = jnp.pad(gidx, (0, pad))
    dst3 = jnp.pad(dst.reshape(NS, NCH_A, KA), ((0, 0), (0, NGRP * 8 - NCH_A), (0, 0)))

    h = x
    layers = [(root0, rel0, bias0), (root1, rel1, bias1), (root2, rel2, bias2)]
    u = v = None
    for li, (root, rel, bias) in enumerate(layers):
        tlo, thi, hroot = _transform(h, rel, root, bias.reshape(1, D))
        acc = _agg_k(tlo, thi, gidx_p, dst3, w_p)
        if li < 2:
            h = _combine(hroot, acc)
        else:
            h, u, v = _combine_uv(hroot, acc, att_w1[:D], att_w1[D:],
                                  att_b1.reshape(1, D))

    b2v = jnp.broadcast_to(att_b2.reshape(1), (L,)).astype(jnp.float32)
    u32 = lax.bitcast_convert_type(u.reshape(N, D // 2, 2), jnp.int32)
    v32 = lax.bitcast_convert_type(v.reshape(N, D // 2, 2), jnp.int32)
    w2p = lax.bitcast_convert_type(
        att_w2.reshape(D).astype(jnp.bfloat16).reshape(D // 2, 2), jnp.int32)
    part = _att_k(u32, v32, src, dst, edge_confidence, w2p, b2v)
    pm, pv = _final(h, part.reshape(4, 128),
                    bayes_w, bayes_b.reshape(1, D),
                    mean_w, mean_b.reshape(1, D),
                    var_w, var_b.reshape(1, D))
    return pm, pv
